# Initial kernel scaffold; baseline (speedup 1.0000x reference)
import jax
import jax.numpy as jnp
from jax.experimental import pallas as pl


def kernel(x, edge_attr, c1_Wq, c1_bq, c1_Wk, c1_bk, c1_Wv, c1_bv, c1_We, c1_be, c1_Ws, c1_bs, c2_Wq, c2_bq, c2_Wk, c2_bk, c2_Wv, c2_bv, c2_We, c2_be, c2_Ws, c2_bs, fc1_W, fc1_b, fc2_W, fc2_b, edge_index):
    raise NotImplementedError("write your pallas kernel here")


# speedup vs baseline: 5.3512x; 5.3512x over previous
import functools

import jax
import jax.numpy as jnp
from jax import lax
from jax.experimental import pallas as pl
from jax.experimental.pallas import tpu as pltpu
from jax.experimental.pallas import tpu_sc as plsc

N_NODES = 10000
N_PAD = 10240
N_EDGES = 320000
NC_SC = 2
NS_SC = 16
LANES = 16
EDGE_CHUNK = 80
ROW_W = 128


def _elu(x):
    return jnp.where(x > 0, x, jnp.exp(jnp.minimum(x, 0.0)) - 1.0)


def _stageA_body(x_ref, wq_ref, bq_ref, wk_ref, bk_ref, wv_ref, bv_ref,
                 we_ref, be_ref, ws_ref, bs_ref,
                 qtab_ref, kvtab_ref, skip_ref):
    x = x_ref[...]
    c = wq_ref.shape[1]
    qs = (jnp.dot(x, wq_ref[...], preferred_element_type=jnp.float32)
          + bq_ref[...]) * (1.0 / (c ** 0.5))
    k = jnp.dot(x, wk_ref[...], preferred_element_type=jnp.float32) + bk_ref[...]
    v = jnp.dot(x, wv_ref[...], preferred_element_type=jnp.float32) + bv_ref[...]
    u = jnp.sum(qs * we_ref[...], axis=1, keepdims=True)
    t = jnp.sum(qs * be_ref[...], axis=1, keepdims=True)
    pad = qtab_ref.shape[1] - c - 2
    qtab_ref[...] = jnp.concatenate(
        [qs, u, t, jnp.zeros((x.shape[0], pad), jnp.float32)], axis=1)
    kvpad = kvtab_ref.shape[1] - 2 * c
    kv = [k, v] if kvpad == 0 else [k, v, jnp.zeros((x.shape[0], kvpad), jnp.float32)]
    kvtab_ref[...] = jnp.concatenate(kv, axis=1)
    skip_ref[...] = (jnp.dot(x, ws_ref[...], preferred_element_type=jnp.float32)
                     + bs_ref[...])


def _dense_tables(x, wq, bq, wk, bk, wv, bv, we, be, ws, bs):
    n, _ = x.shape
    c = wq.shape[1]
    blk = 1000
    grid = n // blk
    full = lambda a: pl.BlockSpec(a.shape, lambda i: (0,) * a.ndim)
    return pl.pallas_call(
        _stageA_body,
        grid=(grid,),
        in_specs=[pl.BlockSpec((blk, x.shape[1]), lambda i: (i, 0))] +
                 [full(a) for a in (wq, bq, wk, bk, wv, bv, we, be, ws, bs)],
        out_specs=[pl.BlockSpec((blk, ROW_W), lambda i: (i, 0)),
                   pl.BlockSpec((blk, ROW_W), lambda i: (i, 0)),
                   pl.BlockSpec((blk, c), lambda i: (i, 0))],
        out_shape=[jax.ShapeDtypeStruct((n, ROW_W), jnp.float32),
                   jax.ShapeDtypeStruct((n, ROW_W), jnp.float32),
                   jax.ShapeDtypeStruct((n, c), jnp.float32)],
    )(x, wq, bq, wk, bk, wv, bv, we, be, ws, bs)


def _combine_body(parts_ref, skip_ref, we_ref, be_ref, h_ref):
    c = skip_ref.shape[1]
    acc = parts_ref[0] + parts_ref[1]
    num = acc[:, :c]
    s = acc[:, c:c + 1]
    sa = acc[:, c + 1:c + 2]
    num = num + sa * we_ref[...] + s * be_ref[...]
    h_ref[...] = _elu(num / (s + 1e-16) + skip_ref[...])


def _combine(parts, skip, we, be):
    n, c = skip.shape
    dpad = parts.shape[2]
    blk = 1000
    full = lambda a: pl.BlockSpec(a.shape, lambda i: (0,) * a.ndim)
    return pl.pallas_call(
        _combine_body,
        grid=(n // blk,),
        in_specs=[pl.BlockSpec((2, blk, dpad), lambda i: (0, i, 0)),
                  pl.BlockSpec((blk, c), lambda i: (i, 0)),
                  full(we), full(be)],
        out_specs=pl.BlockSpec((blk, c), lambda i: (i, 0)),
        out_shape=jax.ShapeDtypeStruct((n, c), jnp.float32),
    )(parts, skip, we, be)


def _head_body(h_ref, w1_ref, b1_ref, w2_ref, b2_ref, out_ref):
    z = _elu(jnp.dot(h_ref[...], w1_ref[...],
                     preferred_element_type=jnp.float32) + b1_ref[...])
    z = jnp.dot(z, w2_ref[...], preferred_element_type=jnp.float32) + b2_ref[...]
    m = jnp.max(z, axis=1, keepdims=True)
    lse = jnp.log(jnp.sum(jnp.exp(z - m), axis=1, keepdims=True)) + m
    out_ref[...] = z - lse


def _head(h, w1, b1, w2, b2):
    n = h.shape[0]
    nc = w2.shape[1]
    blk = 1000
    full = lambda a: pl.BlockSpec(a.shape, lambda i: (0,) * a.ndim)
    return pl.pallas_call(
        _head_body,
        grid=(n // blk,),
        in_specs=[pl.BlockSpec((blk, h.shape[1]), lambda i: (i, 0)),
                  full(w1), full(b1), full(w2), full(b2)],
        out_specs=pl.BlockSpec((blk, nc), lambda i: (i, 0)),
        out_shape=jax.ShapeDtypeStruct((n, nc), jnp.float32),
    )(h, w1, b1, w2, b2)


def _edge_pass_body(c_dim,
                    qtab_hbm, kvtab_hbm, src_hbm, dst_hbm, attr_hbm, zeros_hbm,
                    parts_hbm,
                    src_v, dst_v, attr_v, q_rows, kv_rows, msg_v, acc_shared,
                    sem0, sem1):
    cid = lax.axis_index("c")
    sid = lax.axis_index("s")
    wid = sid * NC_SC + cid
    epw = N_EDGES // (NC_SC * NS_SC)
    nchunks = epw // EDGE_CHUNK
    rows_per_sub = N_PAD // NS_SC

    pltpu.sync_copy(zeros_hbm.at[pl.ds(sid * rows_per_sub, rows_per_sub)],
                    acc_shared.at[pl.ds(sid * rows_per_sub, rows_per_sub)])
    plsc.subcore_barrier()

    def chunk_body(i, carry):
        base = wid * epw + i * EDGE_CHUNK
        pltpu.sync_copy(src_hbm.at[pl.ds(base, EDGE_CHUNK)], src_v)
        pltpu.sync_copy(dst_hbm.at[pl.ds(base, EDGE_CHUNK)], dst_v)
        pltpu.sync_copy(attr_hbm.at[pl.ds(base, EDGE_CHUNK)], attr_v)
        gkv = pltpu.async_copy(kvtab_hbm.at[src_v], kv_rows, sem0)
        gq = pltpu.async_copy(qtab_hbm.at[dst_v], q_rows, sem1)
        gkv.wait()
        gq.wait()

        def group_body(g, carry2):
            rows = g * LANES + lax.iota(jnp.int32, LANES)
            attr = attr_v[pl.ds(g * LANES, LANES)]
            alpha = plsc.load_gather(q_rows, [rows, jnp.full((LANES,), c_dim, jnp.int32)]) * attr
            alpha = alpha + plsc.load_gather(
                q_rows, [rows, jnp.full((LANES,), c_dim + 1, jnp.int32)])
            for c in range(c_dim):
                colc = jnp.full((LANES,), c, jnp.int32)
                qc = plsc.load_gather(q_rows, [rows, colc])
                kc = plsc.load_gather(kv_rows, [rows, colc])
                alpha = alpha + qc * kc
            s = jnp.exp(alpha)
            for c in range(c_dim):
                vc = plsc.load_gather(
                    kv_rows, [rows, jnp.full((LANES,), c_dim + c, jnp.int32)])
                plsc.store_scatter(msg_v, [rows, jnp.full((LANES,), c, jnp.int32)],
                                   vc * s)
            plsc.store_scatter(msg_v, [rows, jnp.full((LANES,), c_dim, jnp.int32)], s)
            plsc.store_scatter(msg_v,
                               [rows, jnp.full((LANES,), c_dim + 1, jnp.int32)],
                               s * attr)
            return carry2

        lax.fori_loop(0, EDGE_CHUNK // LANES, group_body, 0)
        pltpu.sync_copy(msg_v, acc_shared.at[dst_v], add=True)
        return carry

    lax.fori_loop(0, nchunks, chunk_body, 0)
    plsc.subcore_barrier()

    pltpu.sync_copy(acc_shared.at[pl.ds(sid * rows_per_sub, rows_per_sub)],
                    parts_hbm.at[cid, pl.ds(sid * rows_per_sub, rows_per_sub)])


def _edge_pass(qtab, kvtab, src, dst, attr, c_dim):
    mesh = plsc.VectorSubcoreMesh(core_axis_name="c", subcore_axis_name="s")
    zeros = jnp.zeros((N_PAD, ROW_W), jnp.float32)
    body = functools.partial(_edge_pass_body, c_dim)
    fn = pl.kernel(
        body,
        out_type=jax.ShapeDtypeStruct((NC_SC, N_PAD, ROW_W), jnp.float32),
        mesh=mesh,
        scratch_types=[
            pltpu.VMEM((EDGE_CHUNK,), jnp.int32),
            pltpu.VMEM((EDGE_CHUNK,), jnp.int32),
            pltpu.VMEM((EDGE_CHUNK,), jnp.float32),
            pltpu.VMEM((EDGE_CHUNK, ROW_W), jnp.float32),
            pltpu.VMEM((EDGE_CHUNK, ROW_W), jnp.float32),
            pltpu.VMEM((EDGE_CHUNK, ROW_W), jnp.float32),
            pltpu.VMEM_SHARED((N_PAD, ROW_W), jnp.float32),
            pltpu.SemaphoreType.DMA,
            pltpu.SemaphoreType.DMA,
        ],
        compiler_params=pltpu.CompilerParams(needs_layout_passes=False),
    )
    return fn(qtab, kvtab, src, dst, attr, zeros)[:, :N_NODES]


def kernel(x, edge_attr, c1_Wq, c1_bq, c1_Wk, c1_bk, c1_Wv, c1_bv, c1_We,
           c1_be, c1_Ws, c1_bs, c2_Wq, c2_bq, c2_Wk, c2_bk, c2_Wv, c2_bv,
           c2_We, c2_be, c2_Ws, c2_bs, fc1_W, fc1_b, fc2_W, fc2_b, edge_index):
    src = edge_index[0]
    dst = edge_index[1]
    attr = edge_attr[:, 0]
    r2 = lambda b: b.reshape(1, -1)

    qtab1, kvtab1, skip1 = _dense_tables(
        x, c1_Wq, r2(c1_bq), c1_Wk, r2(c1_bk), c1_Wv, r2(c1_bv), c1_We,
        r2(c1_be), c1_Ws, r2(c1_bs))
    parts1 = _edge_pass(qtab1, kvtab1, src, dst, attr, 32)
    h1 = _combine(parts1, skip1, c1_We, r2(c1_be))

    qtab2, kvtab2, skip2 = _dense_tables(
        h1, c2_Wq, r2(c2_bq), c2_Wk, r2(c2_bk), c2_Wv, r2(c2_bv), c2_We,
        r2(c2_be), c2_Ws, r2(c2_bs))
    parts2 = _edge_pass(qtab2, kvtab2, src, dst, attr, 64)
    h2 = _combine(parts2, skip2, c2_We, r2(c2_be))

    return _head(h2, fc1_W, r2(fc1_b), fc2_W, r2(fc2_b))
